# baseline (speedup 1.0000x reference)
import functools

import jax
import jax.numpy as jnp
from jax import lax
from jax.experimental import pallas as pl
from jax.experimental.pallas import tpu as pltpu
from jax.experimental.pallas import tpu_sc as plsc

N_NODES = 3_000_000
E_EDGES = 3_200_000
NGRAM = 5

CN = 1_536_000
DUMP = 8_192
ACC = CN + DUMP
ZSLICE = ACC // 16
BLKE = 5_120
NBLK = E_EDGES // BLKE
NBQ, NBR = divmod(NBLK, 16)
NBS = NBQ + 1
WB = CN // 16

DEDGE = 64_000
DGRID = E_EDGES // DEDGE


def _data_body(w_ref, f_ref, v_ref, o_ref):
    s = jnp.sum(f_ref[...] * w_ref[...], axis=0, keepdims=True)
    o_ref[0] = s * v_ref[0]


def _compute_data(features, values, a0_weight):
    ft = features.T
    v3d = values.reshape(DGRID, 1, DEDGE)
    out = pl.pallas_call(
        _data_body,
        grid=(DGRID,),
        in_specs=[
            pl.BlockSpec((NGRAM, 1), lambda i: (0, 0)),
            pl.BlockSpec((NGRAM, DEDGE), lambda i: (0, i)),
            pl.BlockSpec((1, 1, DEDGE), lambda i: (i, 0, 0)),
        ],
        out_specs=pl.BlockSpec((1, 1, DEDGE), lambda i: (i, 0, 0)),
        out_shape=jax.ShapeDtypeStruct((DGRID, 1, DEDGE), jnp.float32),
    )(a0_weight.reshape(NGRAM, 1), ft, v3d)
    return out.reshape(E_EDGES)


def _sc_scatter_body(rows_hbm, cols_hbm, data_hbm, zeros_hbm,
                     ind_hbm, mem_hbm, acc, nbuf, dbuf, ibuf):
    c = lax.axis_index("c")
    s = lax.axis_index("s")
    nb = jnp.where(s < NBR, NBQ + 1, NBQ)
    start = s * NBQ + jnp.minimum(s, NBR)

    def scan(src_hbm, p, ind_row):
        def body(i, carry):
            valid = i < nb
            base = jnp.where(valid, start + i, 0) * BLKE
            vbias = jnp.where(valid, 0, 1 << 24)
            pltpu.sync_copy(src_hbm.at[pl.ds(base, BLKE)], nbuf)
            pltpu.sync_copy(data_hbm.at[pl.ds(base, BLKE)], dbuf)
            if p == 0:
                pltpu.sync_copy(nbuf, ind_hbm.at[ind_row, pl.ds(base, BLKE)])

            def chunk(cj, carry2):
                for u in range(16):
                    off = cj * 256 + u * 16
                    r = nbuf[pl.ds(off, 16)]
                    if p == 0:
                        rr = r + vbias
                        sel = rr < CN
                        loc = rr
                    else:
                        rr = r - vbias
                        sel = rr >= CN
                        loc = rr - CN
                    ibuf[pl.ds(off, 16)] = jnp.where(
                        sel, loc, CN + jnp.bitwise_and(rr, DUMP - 1))
                return carry2
            lax.fori_loop(0, BLKE // 256, chunk, 0)
            pltpu.sync_copy(dbuf, acc.at[ibuf], add=True)
            return carry
        lax.fori_loop(0, NBS, body, 0)

    for p in range(2):
        pltpu.sync_copy(zeros_hbm, acc.at[pl.ds(s * ZSLICE, ZSLICE)])
        plsc.subcore_barrier()

        @pl.when(c == 0)
        def _():
            scan(rows_hbm, p, 0)

        @pl.when(c == 1)
        def _():
            scan(cols_hbm, p, 1)

        plsc.subcore_barrier()

        pltpu.sync_copy(acc.at[pl.ds(s * WB, WB)],
                        mem_hbm.at[pl.ds(c * 2 * CN + p * CN + s * WB, WB)])

        plsc.subcore_barrier()


_sc_scatter = functools.partial(
    pl.kernel,
    out_type=(jax.ShapeDtypeStruct((2, E_EDGES), jnp.int32),
              jax.ShapeDtypeStruct((4 * CN,), jnp.float32)),
    mesh=plsc.VectorSubcoreMesh(core_axis_name="c", subcore_axis_name="s"),
    scratch_types=[
        pltpu.VMEM_SHARED((ACC,), jnp.float32),
        pltpu.VMEM((BLKE,), jnp.int32),
        pltpu.VMEM((BLKE,), jnp.float32),
        pltpu.VMEM((BLKE,), jnp.int32),
    ],
)(_sc_scatter_body)


def kernel(values, features, rows, cols, a0_weight):
    data = _compute_data(features, values, a0_weight)
    zeros = jnp.zeros((ZSLICE,), jnp.float32)
    indices, mem_flat = _sc_scatter(rows, cols, data, zeros)
    mem = mem_flat.reshape(2, 2 * CN)[:, :N_NODES].T
    return (indices, data, mem)


# speedup vs baseline: 1.5572x; 1.5572x over previous
import functools

import jax
import jax.numpy as jnp
from jax import lax
from jax.experimental import pallas as pl
from jax.experimental.pallas import tpu as pltpu
from jax.experimental.pallas import tpu_sc as plsc

N_NODES = 3_000_000
E_EDGES = 3_200_000
NGRAM = 5

CN = 1_536_000
DUMP = 8_192
ACC = CN + DUMP
ZSLICE = ACC // 16
BLKE = 5_120
NBLK = E_EDGES // BLKE
NBQ, NBR = divmod(NBLK, 16)
NBS = NBQ + 1
WB = CN // 16

DEDGE = 64_000
DGRID = E_EDGES // DEDGE


def _data_body(w_ref, f_ref, v_ref, o_ref):
    s = jnp.sum(f_ref[...] * w_ref[...], axis=0, keepdims=True)
    o_ref[0] = s * v_ref[0]


def _compute_data(features, values, a0_weight):
    ft = features.T
    v3d = values.reshape(DGRID, 1, DEDGE)
    out = pl.pallas_call(
        _data_body,
        grid=(DGRID,),
        in_specs=[
            pl.BlockSpec((NGRAM, 1), lambda i: (0, 0)),
            pl.BlockSpec((NGRAM, DEDGE), lambda i: (0, i)),
            pl.BlockSpec((1, 1, DEDGE), lambda i: (i, 0, 0)),
        ],
        out_specs=pl.BlockSpec((1, 1, DEDGE), lambda i: (i, 0, 0)),
        out_shape=jax.ShapeDtypeStruct((DGRID, 1, DEDGE), jnp.float32),
    )(a0_weight.reshape(NGRAM, 1), ft, v3d)
    return out.reshape(E_EDGES)


def _sc_scatter_body(rows_hbm, cols_hbm, data_hbm, zeros_hbm,
                     ind_hbm, mem0_hbm, mem1_hbm, acc, nbuf, dbuf, ibuf):
    c = lax.axis_index("c")
    s = lax.axis_index("s")
    nb = jnp.where(s < NBR, NBQ + 1, NBQ)
    start = s * NBQ + jnp.minimum(s, NBR)

    def scan(src_hbm, p, ind_row):
        def body(i, carry):
            valid = i < nb
            base = jnp.where(valid, start + i, 0) * BLKE
            vbias = jnp.where(valid, 0, 1 << 24)
            pltpu.sync_copy(src_hbm.at[pl.ds(base, BLKE)], nbuf)
            pltpu.sync_copy(data_hbm.at[pl.ds(base, BLKE)], dbuf)
            if p == 0:
                pltpu.sync_copy(nbuf, ind_hbm.at[ind_row, pl.ds(base, BLKE)])

            def chunk(cj, carry2):
                for u in range(16):
                    off = cj * 256 + u * 16
                    r = nbuf[pl.ds(off, 16)]
                    if p == 0:
                        rr = r + vbias
                        sel = rr < CN
                        loc = rr
                    else:
                        rr = r - vbias
                        sel = rr >= CN
                        loc = rr - CN
                    ibuf[pl.ds(off, 16)] = jnp.where(
                        sel, loc, CN + jnp.bitwise_and(rr, DUMP - 1))
                return carry2
            lax.fori_loop(0, BLKE // 256, chunk, 0)
            pltpu.sync_copy(dbuf, acc.at[ibuf], add=True)
            return carry
        lax.fori_loop(0, NBS, body, 0)

    for p in range(2):
        pltpu.sync_copy(zeros_hbm, acc.at[pl.ds(s * ZSLICE, ZSLICE)])
        plsc.subcore_barrier()

        @pl.when(c == 0)
        def _():
            scan(rows_hbm, p, 0)

        @pl.when(c == 1)
        def _():
            scan(cols_hbm, p, 1)

        plsc.subcore_barrier()

        @pl.when(c == 0)
        def _():
            pltpu.sync_copy(acc.at[pl.ds(s * WB, WB)],
                            mem0_hbm.at[pl.ds(p * CN + s * WB, WB)])

        @pl.when(c == 1)
        def _():
            pltpu.sync_copy(acc.at[pl.ds(s * WB, WB)],
                            mem1_hbm.at[pl.ds(p * CN + s * WB, WB)])

        plsc.subcore_barrier()


_sc_scatter = functools.partial(
    pl.kernel,
    out_type=(jax.ShapeDtypeStruct((2, E_EDGES), jnp.int32),
              jax.ShapeDtypeStruct((2 * CN,), jnp.float32),
              jax.ShapeDtypeStruct((2 * CN,), jnp.float32)),
    mesh=plsc.VectorSubcoreMesh(core_axis_name="c", subcore_axis_name="s"),
    scratch_types=[
        pltpu.VMEM_SHARED((ACC,), jnp.float32),
        pltpu.VMEM((BLKE,), jnp.int32),
        pltpu.VMEM((BLKE,), jnp.float32),
        pltpu.VMEM((BLKE,), jnp.int32),
    ],
)(_sc_scatter_body)


def kernel(values, features, rows, cols, a0_weight):
    data = _compute_data(features, values, a0_weight)
    zeros = jnp.zeros((ZSLICE,), jnp.float32)
    indices, mem0, mem1 = _sc_scatter(rows, cols, data, zeros)
    mem = jnp.stack([mem0[:N_NODES], mem1[:N_NODES]], axis=1)
    return (indices, data, mem)


# speedup vs baseline: 1.7418x; 1.1185x over previous
import functools

import jax
import jax.numpy as jnp
from jax import lax
from jax.experimental import pallas as pl
from jax.experimental.pallas import tpu as pltpu
from jax.experimental.pallas import tpu_sc as plsc

N_NODES = 3_000_000
E_EDGES = 3_200_000
NGRAM = 5

CN = 1_536_000
DUMP = 8_192
ACC = CN + DUMP
ZSLICE = ACC // 16
BLKE = 5_120
NBLK = E_EDGES // BLKE
NBQ, NBR = divmod(NBLK, 16)
NBS = NBQ + 1
WB = CN // 16

DEDGE = 64_000
DGRID = E_EDGES // DEDGE


def _data_body(w_ref, f_ref, v_ref, o_ref):
    s = jnp.sum(f_ref[...] * w_ref[...], axis=0, keepdims=True)
    o_ref[0] = s * v_ref[0]


def _compute_data(features, values, a0_weight):
    ft = features.T
    v3d = values.reshape(DGRID, 1, DEDGE)
    out = pl.pallas_call(
        _data_body,
        grid=(DGRID,),
        in_specs=[
            pl.BlockSpec((NGRAM, 1), lambda i: (0, 0)),
            pl.BlockSpec((NGRAM, DEDGE), lambda i: (0, i)),
            pl.BlockSpec((1, 1, DEDGE), lambda i: (i, 0, 0)),
        ],
        out_specs=pl.BlockSpec((1, 1, DEDGE), lambda i: (i, 0, 0)),
        out_shape=jax.ShapeDtypeStruct((DGRID, 1, DEDGE), jnp.float32),
    )(a0_weight.reshape(NGRAM, 1), ft, v3d)
    return out.reshape(E_EDGES)


def _sc_scatter_body(rows_hbm, cols_hbm, data_hbm, zeros_hbm,
                     ind_hbm, mem3_hbm, acc, nbuf, dbuf, ibuf, wsem):
    c = lax.axis_index("c")
    s = lax.axis_index("s")
    nb = jnp.where(s < NBR, NBQ + 1, NBQ)
    start = s * NBQ + jnp.minimum(s, NBR)

    def scan(src_hbm, p, ind_row):
        def body(i, carry):
            valid = i < nb
            base = jnp.where(valid, start + i, 0) * BLKE
            vbias = jnp.where(valid, 0, 1 << 24)
            pltpu.sync_copy(src_hbm.at[pl.ds(base, BLKE)], nbuf)
            pltpu.sync_copy(data_hbm.at[pl.ds(base, BLKE)], dbuf)
            if p == 0:
                pltpu.sync_copy(nbuf, ind_hbm.at[ind_row, pl.ds(base, BLKE)])

            def chunk(cj, carry2):
                for u in range(16):
                    off = cj * 256 + u * 16
                    r = nbuf[pl.ds(off, 16)]
                    if p == 0:
                        rr = r + vbias
                        sel = rr < CN
                        loc = rr
                    else:
                        rr = r - vbias
                        sel = rr >= CN
                        loc = rr - CN
                    ibuf[pl.ds(off, 16)] = jnp.where(
                        sel, loc, CN + jnp.bitwise_and(rr, DUMP - 1))
                return carry2
            lax.fori_loop(0, BLKE // 256, chunk, 0)
            pltpu.sync_copy(dbuf, acc.at[ibuf], add=True)
            return carry
        lax.fori_loop(0, NBS, body, 0)

    for p in range(2):
        pltpu.sync_copy(zeros_hbm, acc.at[pl.ds(s * ZSLICE, ZSLICE)])
        plsc.subcore_barrier()

        @pl.when(c == 0)
        def _():
            scan(rows_hbm, p, 0)

        @pl.when(c == 1)
        def _():
            scan(cols_hbm, p, 1)

        plsc.subcore_barrier()

        gbase = p * (CN // 128) + s * (WB // 128)

        def wb(g, carry):
            pltpu.async_copy(acc.at[pl.ds(s * WB + g * 128, 128)],
                             mem3_hbm.at[gbase + g, c], wsem)
            return carry
        lax.fori_loop(0, WB // 128, wb, 0)
        pltpu.make_async_copy(data_hbm.at[pl.ds(0, WB)],
                              acc.at[pl.ds(s * WB, WB)], wsem).wait()

        plsc.subcore_barrier()


_sc_scatter = functools.partial(
    pl.kernel,
    out_type=(jax.ShapeDtypeStruct((2, E_EDGES), jnp.int32),
              jax.ShapeDtypeStruct((2 * CN // 128, 2, 128), jnp.float32)),
    mesh=plsc.VectorSubcoreMesh(core_axis_name="c", subcore_axis_name="s"),
    scratch_types=[
        pltpu.VMEM_SHARED((ACC,), jnp.float32),
        pltpu.VMEM((BLKE,), jnp.int32),
        pltpu.VMEM((BLKE,), jnp.float32),
        pltpu.VMEM((BLKE,), jnp.int32),
        pltpu.SemaphoreType.DMA,
    ],
)(_sc_scatter_body)


def kernel(values, features, rows, cols, a0_weight):
    data = _compute_data(features, values, a0_weight)
    zeros = jnp.zeros((ZSLICE,), jnp.float32)
    indices, mem3 = _sc_scatter(rows, cols, data, zeros)
    mem = mem3.transpose(0, 2, 1).reshape(2 * CN, 2)[:N_NODES]
    return (indices, data, mem)


# speedup vs baseline: 2.3178x; 1.3307x over previous
import functools

import jax
import jax.numpy as jnp
from jax import lax
from jax.experimental import pallas as pl
from jax.experimental.pallas import tpu as pltpu
from jax.experimental.pallas import tpu_sc as plsc

N_NODES = 3_000_000
E_EDGES = 3_200_000
NGRAM = 5

CN = 1_536_000
DUMP = 8_192
ACC = CN + DUMP
ZSLICE = ACC // 16
BLKE = 5_120
NBLK = E_EDGES // BLKE
NBQ, NBR = divmod(NBLK, 16)
NBS = NBQ + 1
WB = CN // 16

DEDGE = 64_000
DGRID = E_EDGES // DEDGE


def _data_body(w_ref, f_ref, v_ref, o_ref):
    s = jnp.sum(f_ref[...] * w_ref[...], axis=0, keepdims=True)
    o_ref[0] = s * v_ref[0]


def _compute_data(features, values, a0_weight):
    ft = features.T
    v3d = values.reshape(DGRID, 1, DEDGE)
    out = pl.pallas_call(
        _data_body,
        grid=(DGRID,),
        in_specs=[
            pl.BlockSpec((NGRAM, 1), lambda i: (0, 0)),
            pl.BlockSpec((NGRAM, DEDGE), lambda i: (0, i)),
            pl.BlockSpec((1, 1, DEDGE), lambda i: (i, 0, 0)),
        ],
        out_specs=pl.BlockSpec((1, 1, DEDGE), lambda i: (i, 0, 0)),
        out_shape=jax.ShapeDtypeStruct((DGRID, 1, DEDGE), jnp.float32),
    )(a0_weight.reshape(NGRAM, 1), ft, v3d)
    return out.reshape(E_EDGES)


def _sc_scatter_body(rows_hbm, cols_hbm, data_hbm, zeros_hbm,
                     ind_hbm, mem3_hbm, acc, nbufa, dbufa, nbufb, dbufb,
                     ibuf, wsem, sema, semb):
    c = lax.axis_index("c")
    s = lax.axis_index("s")
    nb = jnp.where(s < NBR, NBQ + 1, NBQ)
    start = s * NBQ + jnp.minimum(s, NBR)

    def scan(src_hbm, p, ind_row):
        def fire(i, nbuf, dbuf, sem):
            base = jnp.where(i < nb, start + i, 0) * BLKE
            pltpu.async_copy(src_hbm.at[pl.ds(base, BLKE)], nbuf, sem)
            pltpu.async_copy(data_hbm.at[pl.ds(base, BLKE)], dbuf, sem)

        def proc(i, nbuf, dbuf, sem):
            pltpu.make_async_copy(
                src_hbm.at[pl.ds(0, BLKE)], nbuf, sem).wait()
            pltpu.make_async_copy(
                data_hbm.at[pl.ds(0, BLKE)], dbuf, sem).wait()
            valid = i < nb
            base = jnp.where(valid, start + i, 0) * BLKE
            vbias = jnp.where(valid, 0, 1 << 24)
            if p == 0:
                pltpu.sync_copy(nbuf, ind_hbm.at[ind_row, pl.ds(base, BLKE)])

            def chunk(cj, carry2):
                for u in range(16):
                    off = cj * 256 + u * 16
                    r = nbuf[pl.ds(off, 16)]
                    if p == 0:
                        rr = r + vbias
                        sel = rr < CN
                        loc = rr
                    else:
                        rr = r - vbias
                        sel = rr >= CN
                        loc = rr - CN
                    ibuf[pl.ds(off, 16)] = jnp.where(
                        sel, loc, CN + jnp.bitwise_and(rr, DUMP - 1))
                return carry2
            lax.fori_loop(0, BLKE // 256, chunk, 0)
            pltpu.sync_copy(dbuf, acc.at[ibuf], add=True)

        fire(0, nbufa, dbufa, sema)

        def body2(h, carry):
            i0 = 2 * h
            fire(i0 + 1, nbufb, dbufb, semb)
            proc(i0, nbufa, dbufa, sema)

            @pl.when(h < NBS // 2 - 1)
            def _():
                fire(i0 + 2, nbufa, dbufa, sema)

            proc(i0 + 1, nbufb, dbufb, semb)
            return carry
        lax.fori_loop(0, NBS // 2, body2, 0)

    for p in range(2):
        pltpu.sync_copy(zeros_hbm, acc.at[pl.ds(s * ZSLICE, ZSLICE)])
        plsc.subcore_barrier()

        @pl.when(c == 0)
        def _():
            scan(rows_hbm, p, 0)

        @pl.when(c == 1)
        def _():
            scan(cols_hbm, p, 1)

        plsc.subcore_barrier()

        gbase = p * (CN // 128) + s * (WB // 128)

        def wb(g, carry):
            pltpu.async_copy(acc.at[pl.ds(s * WB + g * 128, 128)],
                             mem3_hbm.at[gbase + g, c], wsem)
            return carry
        lax.fori_loop(0, WB // 128, wb, 0)
        pltpu.make_async_copy(data_hbm.at[pl.ds(0, WB)],
                              acc.at[pl.ds(s * WB, WB)], wsem).wait()

        plsc.subcore_barrier()


_sc_scatter = functools.partial(
    pl.kernel,
    out_type=(jax.ShapeDtypeStruct((2, E_EDGES), jnp.int32),
              jax.ShapeDtypeStruct((2 * CN // 128, 2, 128), jnp.float32)),
    mesh=plsc.VectorSubcoreMesh(core_axis_name="c", subcore_axis_name="s"),
    scratch_types=[
        pltpu.VMEM_SHARED((ACC,), jnp.float32),
        pltpu.VMEM((BLKE,), jnp.int32),
        pltpu.VMEM((BLKE,), jnp.float32),
        pltpu.VMEM((BLKE,), jnp.int32),
        pltpu.VMEM((BLKE,), jnp.float32),
        pltpu.VMEM((BLKE,), jnp.int32),
        pltpu.SemaphoreType.DMA,
        pltpu.SemaphoreType.DMA,
        pltpu.SemaphoreType.DMA,
    ],
)(_sc_scatter_body)


def kernel(values, features, rows, cols, a0_weight):
    data = _compute_data(features, values, a0_weight)
    zeros = jnp.zeros((ZSLICE,), jnp.float32)
    indices, mem3 = _sc_scatter(rows, cols, data, zeros)
    mem = mem3.transpose(0, 2, 1).reshape(2 * CN, 2)[:N_NODES]
    return (indices, data, mem)
